# baseline (speedup 1.0000x reference)
import jax
import jax.numpy as jnp
from jax import lax
from jax.experimental import pallas as pl
from jax.experimental.pallas import tpu as pltpu

_TSCALE = 512
_BB = 8


def _prop_mask_kernel(o_ref, buf, sem):
    ts = _TSCALE
    d = lax.broadcasted_iota(jnp.int32, (ts, ts), 0)
    s = lax.broadcasted_iota(jnp.int32, (ts, ts), 1)
    cond = ((d + s) < ts) & (
        (d < ts // 4)
        | ((d < ts // 2) & ((s & 1) == 0))
        | ((s & 3) == 0)
    )
    block = jnp.where(cond, 1.0, 0.0).astype(jnp.float32)
    buf[...] = jnp.broadcast_to(block[None], buf.shape)

    n = o_ref.shape[0] // _BB

    def fire(i, _):
        pltpu.make_async_copy(buf, o_ref.at[pl.ds(i * _BB, _BB)], sem).start()
        return 0

    lax.fori_loop(0, n, fire, 0)

    def drain(i, _):
        pltpu.make_async_copy(buf, o_ref.at[pl.ds(i * _BB, _BB)], sem).wait()
        return 0

    lax.fori_loop(0, n, drain, 0)


def kernel(start, end, actionness):
    B = start.shape[0]
    ts = _TSCALE
    return pl.pallas_call(
        _prop_mask_kernel,
        out_specs=pl.BlockSpec(memory_space=pl.ANY),
        out_shape=jax.ShapeDtypeStruct((B, ts, ts), jnp.float32),
        scratch_shapes=[
            pltpu.VMEM((_BB, ts, ts), jnp.float32),
            pltpu.SemaphoreType.DMA,
        ],
    )()


# speedup vs baseline: 1.0786x; 1.0786x over previous
import jax
import jax.numpy as jnp
from jax.experimental import pallas as pl

_TSCALE = 512


def _prop_mask_kernel(o_ref):
    ts = _TSCALE
    d = jax.lax.broadcasted_iota(jnp.int32, (ts, ts), 0)
    s = jax.lax.broadcasted_iota(jnp.int32, (ts, ts), 1)
    cond = ((d + s) < ts) & (
        (d < ts // 4)
        | ((d < ts // 2) & ((s & 1) == 0))
        | ((s & 3) == 0)
    )
    block = jnp.where(cond, 1.0, 0.0).astype(jnp.float32)
    o_ref[...] = jnp.broadcast_to(block[None], o_ref.shape)


def kernel(start, end, actionness):
    B = start.shape[0]
    ts = _TSCALE
    bb = 4
    return pl.pallas_call(
        _prop_mask_kernel,
        grid=(B // bb,),
        out_specs=pl.BlockSpec((bb, ts, ts), lambda i: (i, 0, 0)),
        out_shape=jax.ShapeDtypeStruct((B, ts, ts), jnp.float32),
    )()
